# baseline (speedup 1.0000x reference)
import jax
import jax.numpy as jnp
from jax import lax
from jax.experimental import pallas as pl
from jax.experimental.pallas import tpu as pltpu
from jax.experimental.pallas import tpu_sc as plsc

B = 16384
D = 32
NC = 2
NS = 16
NW = NC * NS
BPW = B // NW
L = 16
SG = 2
K = L // SG
GROUPS = BPW // L

_LN2 = 0.6931471805599453
_C1 = _LN2 / (1 << 23)
_C2 = 126.94269504 * _LN2


def _softplus16(x):
    w = jnp.exp(-jnp.abs(x))
    z = 1.0 + w
    zb = plsc.bitcast(z, jnp.int32)
    y0 = zb.astype(jnp.float32) * _C1 - _C2
    t = z * jnp.exp(-y0) - 1.0
    corr = t * (1.0 + t * (-0.5 + t * (1.0 / 3.0 + t * -0.25)))
    return jnp.maximum(x, 0.0) + y0 + corr


_IOTA = None


def _body(user_h, item_h, theta_h, a_h, b_h, out_h,
          uidx_v, iidx_v, th_blk, a_blk, b_blk, o_v, sp_blk,
          hsem, xsem, bsem):
    wid = lax.axis_index("s") * NC + lax.axis_index("c")
    base = wid * BPW
    pltpu.sync_copy(user_h.at[pl.ds(base, BPW)], uidx_v)
    pltpu.sync_copy(item_h.at[pl.ds(base, BPW)], iidx_v)

    iota = lax.iota(jnp.int32, L)
    sid = lax.axis_index("s")

    def fetch(k, ublk16, iblk16):
        for j in range(SG):
            e = k * SG + j
            ub = ublk16[e]
            ib = iblk16[e]
            pltpu.async_copy(
                theta_h.at[:, pl.ds(pl.multiple_of(ub * 128, 128), 128)],
                sp_blk.at[sid, k % 2, j, 0], hsem.at[k % 2])
            pltpu.async_copy(
                a_h.at[:, pl.ds(pl.multiple_of(ib * 128, 128), 128)],
                sp_blk.at[sid, k % 2, j, 1], hsem.at[k % 2])

    def wait_fetch(k):
        for j in range(SG):
            pltpu.make_async_copy(
                theta_h.at[:, pl.ds(0, 128)], sp_blk.at[sid, k % 2, j, 0],
                hsem.at[k % 2]).wait()
            pltpu.make_async_copy(
                a_h.at[:, pl.ds(0, 128)], sp_blk.at[sid, k % 2, j, 1],
                hsem.at[k % 2]).wait()

    def cross(k):
        for j in range(SG):
            pltpu.async_copy(
                sp_blk.at[sid, k % 2, j, 0], th_blk.at[k % 2, j],
                xsem.at[k % 2])
            pltpu.async_copy(
                sp_blk.at[sid, k % 2, j, 1], a_blk.at[k % 2, j],
                xsem.at[k % 2])

    def wait_cross(k):
        for j in range(SG):
            pltpu.make_async_copy(
                theta_h.at[:, pl.ds(0, 128)], th_blk.at[k % 2, j],
                xsem.at[k % 2]).wait()
            pltpu.make_async_copy(
                a_h.at[:, pl.ds(0, 128)], a_blk.at[k % 2, j],
                xsem.at[k % 2]).wait()

    def issue_b(bq16):
        pltpu.async_copy(b_h.at[bq16], b_blk, bsem)

    def load_vecs(g):
        u16 = uidx_v[pl.ds(g * L, L)]
        i16 = iidx_v[pl.ds(g * L, L)]
        return u16, i16

    u16_0, i16_0 = load_vecs(0)
    fetch(0, jnp.right_shift(u16_0, 7), jnp.right_shift(i16_0, 7))
    fetch(1, jnp.right_shift(u16_0, 7), jnp.right_shift(i16_0, 7))
    issue_b(jnp.right_shift(i16_0, 7))

    def group(g, _):
        u16, i16 = load_vecs(g)
        ublk16 = jnp.right_shift(u16, 7)
        iblk16 = jnp.right_shift(i16, 7)
        ucol16 = jnp.bitwise_and(u16, 127)
        icol16 = jnp.bitwise_and(i16, 127)
        bcol16 = jnp.bitwise_and(i16, 127)
        gn = jnp.minimum(g + 1, GROUPS - 1)
        u16n, i16n = load_vecs(gn)
        ublk16n = jnp.right_shift(u16n, 7)
        iblk16n = jnp.right_shift(i16n, 7)
        not_last = g < GROUPS - 1

        def compute(k, acc):
            for j in range(SG):
                e = k * SG + j
                uc = jnp.full((L,), 1, jnp.int32) * ucol16[e]
                ic = jnp.full((L,), 1, jnp.int32) * icol16[e]
                th_lo = plsc.load_gather(th_blk.at[k % 2, j], [iota, uc])
                th_hi = plsc.load_gather(th_blk.at[k % 2, j], [iota + L, uc])
                a_lo = plsc.load_gather(a_blk.at[k % 2, j], [iota, ic])
                a_hi = plsc.load_gather(a_blk.at[k % 2, j], [iota + L, ic])
                val = _softplus16(a_lo) * th_lo + _softplus16(a_hi) * th_hi
                acc = jnp.where(iota == e, jnp.sum(val), acc)
            return acc

        s16 = jnp.zeros((L,), jnp.float32)
        wait_fetch(0)
        cross(0)
        wait_fetch(1)
        cross(1)
        for k in range(K):
            wait_cross(k)
            s16 = compute(k, s16)
            if k + 2 < K:
                fetch(k + 2, ublk16, iblk16)
            else:
                nk = k + 2 - K

                @pl.when(not_last)
                def _(nk=nk):
                    fetch(nk, ublk16n, iblk16n)

            if 2 <= k + 1 < K:
                wait_fetch(k + 1)
                cross(k + 1)

        pltpu.make_async_copy(b_h.at[pl.ds(0, L)], b_blk, bsem).wait()
        b16 = plsc.load_gather(b_blk, [iota, bcol16])
        o_v[pl.ds(g * L, L)] = 1.0 / (1.0 + jnp.exp(b16 - s16))

        @pl.when(not_last)
        def _():
            issue_b(jnp.right_shift(i16n, 7))

        return 0

    lax.fori_loop(0, GROUPS, group, 0)
    pltpu.sync_copy(o_v, out_h.at[pl.ds(base, BPW)])


@jax.jit
def _mirt_sc(user, item, theta_t, a_t, b128):
    mesh = plsc.VectorSubcoreMesh(
        core_axis_name="c", subcore_axis_name="s", num_cores=NC, num_subcores=NS
    )
    f = pl.kernel(
        _body,
        out_type=jax.ShapeDtypeStruct((B,), jnp.float32),
        mesh=mesh,
        scratch_types=[
            pltpu.VMEM((BPW,), jnp.int32),
            pltpu.VMEM((BPW,), jnp.int32),
            pltpu.VMEM((2, SG, D, 128), jnp.float32),
            pltpu.VMEM((2, SG, D, 128), jnp.float32),
            pltpu.VMEM((L, 128), jnp.float32),
            pltpu.VMEM((BPW,), jnp.float32),
            pltpu.VMEM_SHARED((NS, 2, SG, 2, D, 128), jnp.float32),
            pltpu.SemaphoreType.DMA((2,)),
            pltpu.SemaphoreType.DMA((2,)),
            pltpu.SemaphoreType.DMA,
        ],
        compiler_params=pltpu.CompilerParams(needs_layout_passes=False),
    )
    return f(user, item, theta_t, a_t, b128)


def kernel(user, item, theta_w, a_w, b_w):
    b_flat = jnp.reshape(b_w, (-1,))
    npad = (-b_flat.shape[0]) % 128
    b128 = jnp.reshape(
        jnp.concatenate([b_flat, jnp.zeros((npad,), b_flat.dtype)]), (-1, 128)
    )
    return _mirt_sc(user, item, theta_w.T, a_w.T, b128)


# speedup vs baseline: 1.2718x; 1.2718x over previous
import jax
import jax.numpy as jnp
from jax import lax
from jax.experimental import pallas as pl
from jax.experimental.pallas import tpu as pltpu
from jax.experimental.pallas import tpu_sc as plsc

B = 16384
D = 32
NC = 2
NS = 16
NW = NC * NS
BPW = B // NW
L = 16
SG = 4
GROUPS = BPW // L

_LN2 = 0.6931471805599453
_C1 = _LN2 / (1 << 23)
_C2 = 126.94269504 * _LN2


def _softplus16(x):
    w = jnp.exp(-jnp.abs(x))
    z = 1.0 + w
    zb = plsc.bitcast(z, jnp.int32)
    y0 = zb.astype(jnp.float32) * _C1 - _C2
    t = z * jnp.exp(-y0) - 1.0
    corr = t * (1.0 + t * (-0.5 + t * (1.0 / 3.0 + t * -0.25)))
    return jnp.maximum(x, 0.0) + y0 + corr


def _body(user_h, item_h, theta_h, a_h, b_h, out_h,
          uidx_v, iidx_v, th_blk, a_blk, b_blk, o_v, sem, bsem):
    wid = lax.axis_index("s") * NC + lax.axis_index("c")
    base = wid * BPW
    pltpu.sync_copy(user_h.at[pl.ds(base, BPW)], uidx_v)
    pltpu.sync_copy(item_h.at[pl.ds(base, BPW)], iidx_v)

    iota = lax.iota(jnp.int32, L)

    def issue(k, ublk16, iblk16):
        for j in range(SG):
            e = k * SG + j
            ub = ublk16[e]
            ib = iblk16[e]
            pltpu.async_copy(
                theta_h.at[:, pl.ds(pl.multiple_of(ub * 128, 128), 128)],
                th_blk.at[k % 2, j], sem.at[k % 2])
            pltpu.async_copy(
                a_h.at[:, pl.ds(pl.multiple_of(ib * 128, 128), 128)],
                a_blk.at[k % 2, j], sem.at[k % 2])

    def issue_b(bq16):
        pltpu.async_copy(b_h.at[bq16], b_blk, bsem)

    def wait_sg(k):
        for j in range(SG):
            pltpu.make_async_copy(
                theta_h.at[:, pl.ds(0, 128)], th_blk.at[k % 2, j],
                sem.at[k % 2]).wait()
            pltpu.make_async_copy(
                a_h.at[:, pl.ds(0, 128)], a_blk.at[k % 2, j],
                sem.at[k % 2]).wait()

    def load_vecs(g):
        u16 = uidx_v[pl.ds(g * L, L)]
        i16 = iidx_v[pl.ds(g * L, L)]
        return u16, i16

    u16_0, i16_0 = load_vecs(0)
    issue(0, jnp.right_shift(u16_0, 7), jnp.right_shift(i16_0, 7))
    issue(1, jnp.right_shift(u16_0, 7), jnp.right_shift(i16_0, 7))
    issue_b(jnp.right_shift(i16_0, 7))

    def group(g, _):
        u16, i16 = load_vecs(g)
        ublk16 = jnp.right_shift(u16, 7)
        iblk16 = jnp.right_shift(i16, 7)
        ucol16 = jnp.bitwise_and(u16, 127)
        icol16 = jnp.bitwise_and(i16, 127)
        bcol16 = jnp.bitwise_and(i16, 127)
        gn = jnp.minimum(g + 1, GROUPS - 1)
        u16n, i16n = load_vecs(gn)
        ublk16n = jnp.right_shift(u16n, 7)
        iblk16n = jnp.right_shift(i16n, 7)
        not_last = g < GROUPS - 1

        def compute(k, acc):
            for j in range(SG):
                e = k * SG + j
                uc = jnp.full((L,), 1, jnp.int32) * ucol16[e]
                ic = jnp.full((L,), 1, jnp.int32) * icol16[e]
                th_lo = plsc.load_gather(th_blk.at[k % 2, j], [iota, uc])
                th_hi = plsc.load_gather(th_blk.at[k % 2, j], [iota + L, uc])
                a_lo = plsc.load_gather(a_blk.at[k % 2, j], [iota, ic])
                a_hi = plsc.load_gather(a_blk.at[k % 2, j], [iota + L, ic])
                val = _softplus16(a_lo) * th_lo + _softplus16(a_hi) * th_hi
                acc = jnp.where(iota == e, jnp.sum(val), acc)
            return acc

        s16 = jnp.zeros((L,), jnp.float32)
        wait_sg(0)
        s16 = compute(0, s16)
        issue(2, ublk16, iblk16)
        wait_sg(1)
        s16 = compute(1, s16)
        issue(3, ublk16, iblk16)
        wait_sg(2)
        s16 = compute(2, s16)

        @pl.when(not_last)
        def _():
            issue(0, ublk16n, iblk16n)

        wait_sg(3)
        s16 = compute(3, s16)

        @pl.when(not_last)
        def _():
            issue(1, ublk16n, iblk16n)

        pltpu.make_async_copy(b_h.at[pl.ds(0, L)], b_blk, bsem).wait()
        b16 = plsc.load_gather(b_blk, [iota, bcol16])
        o_v[pl.ds(g * L, L)] = 1.0 / (1.0 + jnp.exp(b16 - s16))

        @pl.when(not_last)
        def _():
            issue_b(jnp.right_shift(i16n, 7))

        return 0

    lax.fori_loop(0, GROUPS, group, 0)
    pltpu.sync_copy(o_v, out_h.at[pl.ds(base, BPW)])


@jax.jit
def _mirt_sc(user, item, theta_t, a_t, b128):
    mesh = plsc.VectorSubcoreMesh(
        core_axis_name="c", subcore_axis_name="s", num_cores=NC, num_subcores=NS
    )
    f = pl.kernel(
        _body,
        out_type=jax.ShapeDtypeStruct((B,), jnp.float32),
        mesh=mesh,
        scratch_types=[
            pltpu.VMEM((BPW,), jnp.int32),
            pltpu.VMEM((BPW,), jnp.int32),
            pltpu.VMEM((2, SG, D, 128), jnp.float32),
            pltpu.VMEM((2, SG, D, 128), jnp.float32),
            pltpu.VMEM((L, 128), jnp.float32),
            pltpu.VMEM((BPW,), jnp.float32),
            pltpu.SemaphoreType.DMA((2,)),
            pltpu.SemaphoreType.DMA,
        ],
        compiler_params=pltpu.CompilerParams(needs_layout_passes=False),
    )
    return f(user, item, theta_t, a_t, b128)


def kernel(user, item, theta_w, a_w, b_w):
    b_flat = jnp.reshape(b_w, (-1,))
    npad = (-b_flat.shape[0]) % 128
    b128 = jnp.reshape(
        jnp.concatenate([b_flat, jnp.zeros((npad,), b_flat.dtype)]), (-1, 128)
    )
    return _mirt_sc(user, item, theta_w.T, a_w.T, b128)


# speedup vs baseline: 1.3392x; 1.0529x over previous
import jax
import jax.numpy as jnp
from jax import lax
from jax.experimental import pallas as pl
from jax.experimental.pallas import tpu as pltpu
from jax.experimental.pallas import tpu_sc as plsc

B = 16384
D = 32
NC = 2
NS = 16
NW = NC * NS
BPW = B // NW
L = 16
SG = 4
GROUPS = BPW // L

_LN2 = 0.6931471805599453
_C1 = _LN2 / (1 << 23)
_C2 = 126.94269504 * _LN2


def _softplus16(x):
    w = jnp.exp(-jnp.abs(x))
    z = 1.0 + w
    zb = plsc.bitcast(z, jnp.int32)
    y0 = zb.astype(jnp.float32) * _C1 - _C2
    t = z * jnp.exp(-y0) - 1.0
    corr = t * (1.0 + t * (-0.5 + t * (1.0 / 3.0 + t * -0.25)))
    return jnp.maximum(x, 0.0) + y0 + corr


def _body(user_h, item_h, theta_h, a_h, b_h, out_h,
          uidx_v, iidx_v, th_blk, a_blk, b_blk, o_v, sem, asem, bsem):
    wid = lax.axis_index("s") * NC + lax.axis_index("c")
    base = wid * BPW
    pltpu.sync_copy(user_h.at[pl.ds(base, BPW)], uidx_v)
    pltpu.sync_copy(item_h.at[pl.ds(base, BPW)], iidx_v)

    iota = lax.iota(jnp.int32, L)

    def issue(k, ublk16, iblk16):
        for j in range(SG):
            e = k * SG + j
            ub = ublk16[e]
            ib = iblk16[e]
            pltpu.async_copy(
                theta_h.at[:, pl.ds(pl.multiple_of(ub * 128, 128), 128)],
                th_blk.at[k % 2, j], sem.at[k % 2])
            pltpu.async_copy(
                a_h.at[:, pl.ds(pl.multiple_of(ib * 128, 128), 128)],
                a_blk.at[k % 2, j], asem.at[k % 2])

    def issue_b(bq16):
        pltpu.async_copy(b_h.at[bq16], b_blk, bsem)

    def wait_sg(k):
        for j in range(SG):
            pltpu.make_async_copy(
                theta_h.at[:, pl.ds(0, 128)], th_blk.at[k % 2, j],
                sem.at[k % 2]).wait()
            pltpu.make_async_copy(
                a_h.at[:, pl.ds(0, 128)], a_blk.at[k % 2, j],
                asem.at[k % 2]).wait()

    def load_vecs(g):
        u16 = uidx_v[pl.ds(g * L, L)]
        i16 = iidx_v[pl.ds(g * L, L)]
        return u16, i16

    u16_0, i16_0 = load_vecs(0)
    issue(0, jnp.right_shift(u16_0, 7), jnp.right_shift(i16_0, 7))
    issue(1, jnp.right_shift(u16_0, 7), jnp.right_shift(i16_0, 7))
    issue_b(jnp.right_shift(i16_0, 7))

    def group(g, _):
        u16, i16 = load_vecs(g)
        ublk16 = jnp.right_shift(u16, 7)
        iblk16 = jnp.right_shift(i16, 7)
        ucol16 = jnp.bitwise_and(u16, 127)
        icol16 = jnp.bitwise_and(i16, 127)
        bcol16 = jnp.bitwise_and(i16, 127)
        gn = jnp.minimum(g + 1, GROUPS - 1)
        u16n, i16n = load_vecs(gn)
        ublk16n = jnp.right_shift(u16n, 7)
        iblk16n = jnp.right_shift(i16n, 7)
        not_last = g < GROUPS - 1

        def compute(k, acc):
            for j in range(SG):
                e = k * SG + j
                uc = jnp.full((L,), 1, jnp.int32) * ucol16[e]
                ic = jnp.full((L,), 1, jnp.int32) * icol16[e]
                th_lo = plsc.load_gather(th_blk.at[k % 2, j], [iota, uc])
                th_hi = plsc.load_gather(th_blk.at[k % 2, j], [iota + L, uc])
                a_lo = plsc.load_gather(a_blk.at[k % 2, j], [iota, ic])
                a_hi = plsc.load_gather(a_blk.at[k % 2, j], [iota + L, ic])
                val = _softplus16(a_lo) * th_lo + _softplus16(a_hi) * th_hi
                acc = jnp.where(iota == e, jnp.sum(val), acc)
            return acc

        s16 = jnp.zeros((L,), jnp.float32)
        wait_sg(0)
        s16 = compute(0, s16)
        issue(2, ublk16, iblk16)
        wait_sg(1)
        s16 = compute(1, s16)
        issue(3, ublk16, iblk16)
        wait_sg(2)
        s16 = compute(2, s16)

        @pl.when(not_last)
        def _():
            issue(0, ublk16n, iblk16n)

        wait_sg(3)
        s16 = compute(3, s16)

        @pl.when(not_last)
        def _():
            issue(1, ublk16n, iblk16n)

        pltpu.make_async_copy(b_h.at[pl.ds(0, L)], b_blk, bsem).wait()
        b16 = plsc.load_gather(b_blk, [iota, bcol16])
        o_v[pl.ds(g * L, L)] = 1.0 / (1.0 + jnp.exp(b16 - s16))

        @pl.when(not_last)
        def _():
            issue_b(jnp.right_shift(i16n, 7))

        return 0

    lax.fori_loop(0, GROUPS, group, 0)
    pltpu.sync_copy(o_v, out_h.at[pl.ds(base, BPW)])


@jax.jit
def _mirt_sc(user, item, theta_t, a_t, b128):
    mesh = plsc.VectorSubcoreMesh(
        core_axis_name="c", subcore_axis_name="s", num_cores=NC, num_subcores=NS
    )
    f = pl.kernel(
        _body,
        out_type=jax.ShapeDtypeStruct((B,), jnp.float32),
        mesh=mesh,
        scratch_types=[
            pltpu.VMEM((BPW,), jnp.int32),
            pltpu.VMEM((BPW,), jnp.int32),
            pltpu.VMEM((2, SG, D, 128), jnp.float32),
            pltpu.VMEM((2, SG, D, 128), jnp.float32),
            pltpu.VMEM((L, 128), jnp.float32),
            pltpu.VMEM((BPW,), jnp.float32),
            pltpu.SemaphoreType.DMA((2,)),
            pltpu.SemaphoreType.DMA((2,)),
            pltpu.SemaphoreType.DMA,
        ],
        compiler_params=pltpu.CompilerParams(needs_layout_passes=False),
    )
    return f(user, item, theta_t, a_t, b128)


def kernel(user, item, theta_w, a_w, b_w):
    b_flat = jnp.reshape(b_w, (-1,))
    npad = (-b_flat.shape[0]) % 128
    b128 = jnp.reshape(
        jnp.concatenate([b_flat, jnp.zeros((npad,), b_flat.dtype)]), (-1, 128)
    )
    return _mirt_sc(user, item, theta_w.T, a_w.T, b128)


# speedup vs baseline: 1.3447x; 1.0042x over previous
import jax
import jax.numpy as jnp
from jax import lax
from jax.experimental import pallas as pl
from jax.experimental.pallas import tpu as pltpu
from jax.experimental.pallas import tpu_sc as plsc

B = 16384
D = 32
NC = 2
NS = 16
NW = NC * NS
BPW = B // NW
L = 16
SG = 4
GROUPS = BPW // L

_LN2 = 0.6931471805599453
_C1 = _LN2 / (1 << 23)
_C2 = 126.94269504 * _LN2


def _softplus16(x):
    w = jnp.exp(-jnp.abs(x))
    z = 1.0 + w
    zb = plsc.bitcast(z, jnp.int32)
    y0 = zb.astype(jnp.float32) * _C1 - _C2
    t = z * jnp.exp(-y0) - 1.0
    corr = t * (1.0 + t * (-0.5 + t * (1.0 / 3.0 + t * -0.25)))
    return jnp.maximum(x, 0.0) + y0 + corr


def _body(user_h, item_h, theta_h, a_h, b_h, out_h,
          uidx_v, iidx_v, th_blk, a_blk, b_blk, o_v, sem, asem, bsem):
    wid = lax.axis_index("s") * NC + lax.axis_index("c")
    base = wid * BPW
    pltpu.sync_copy(user_h.at[pl.ds(base, BPW)], uidx_v)
    pltpu.sync_copy(item_h.at[pl.ds(base, BPW)], iidx_v)

    iota = lax.iota(jnp.int32, L)

    def issue(k, ublk16, iblk16):
        for j in range(SG):
            e = k * SG + j
            ub = ublk16[e]
            ib = iblk16[e]
            pltpu.async_copy(
                theta_h.at[:, pl.ds(pl.multiple_of(ub * 128, 128), 128)],
                th_blk.at[k % 2, j], sem.at[k % 2, j])
            pltpu.async_copy(
                a_h.at[:, pl.ds(pl.multiple_of(ib * 128, 128), 128)],
                a_blk.at[k % 2, j], asem.at[k % 2, j])

    def issue_b(bq16):
        pltpu.async_copy(b_h.at[bq16], b_blk, bsem)

    def wait_sg(k):
        for j in range(SG):
            pltpu.make_async_copy(
                theta_h.at[:, pl.ds(0, 128)], th_blk.at[k % 2, j],
                sem.at[k % 2, j]).wait()
            pltpu.make_async_copy(
                a_h.at[:, pl.ds(0, 128)], a_blk.at[k % 2, j],
                asem.at[k % 2, j]).wait()

    def load_vecs(g):
        u16 = uidx_v[pl.ds(g * L, L)]
        i16 = iidx_v[pl.ds(g * L, L)]
        return u16, i16

    u16_0, i16_0 = load_vecs(0)
    issue(0, jnp.right_shift(u16_0, 7), jnp.right_shift(i16_0, 7))
    issue(1, jnp.right_shift(u16_0, 7), jnp.right_shift(i16_0, 7))
    issue_b(jnp.right_shift(i16_0, 7))

    def group(g, _):
        u16, i16 = load_vecs(g)
        ublk16 = jnp.right_shift(u16, 7)
        iblk16 = jnp.right_shift(i16, 7)
        ucol16 = jnp.bitwise_and(u16, 127)
        icol16 = jnp.bitwise_and(i16, 127)
        bcol16 = jnp.bitwise_and(i16, 127)
        gn = jnp.minimum(g + 1, GROUPS - 1)
        u16n, i16n = load_vecs(gn)
        ublk16n = jnp.right_shift(u16n, 7)
        iblk16n = jnp.right_shift(i16n, 7)
        not_last = g < GROUPS - 1

        def compute(k, acc):
            for j in range(SG):
                e = k * SG + j
                uc = jnp.full((L,), 1, jnp.int32) * ucol16[e]
                ic = jnp.full((L,), 1, jnp.int32) * icol16[e]
                th_lo = plsc.load_gather(th_blk.at[k % 2, j], [iota, uc])
                th_hi = plsc.load_gather(th_blk.at[k % 2, j], [iota + L, uc])
                a_lo = plsc.load_gather(a_blk.at[k % 2, j], [iota, ic])
                a_hi = plsc.load_gather(a_blk.at[k % 2, j], [iota + L, ic])
                val = _softplus16(a_lo) * th_lo + _softplus16(a_hi) * th_hi
                acc = jnp.where(iota == e, jnp.sum(val), acc)
            return acc

        s16 = jnp.zeros((L,), jnp.float32)
        wait_sg(0)
        s16 = compute(0, s16)
        issue(2, ublk16, iblk16)
        wait_sg(1)
        s16 = compute(1, s16)
        issue(3, ublk16, iblk16)
        wait_sg(2)
        s16 = compute(2, s16)

        @pl.when(not_last)
        def _():
            issue(0, ublk16n, iblk16n)

        wait_sg(3)
        s16 = compute(3, s16)

        @pl.when(not_last)
        def _():
            issue(1, ublk16n, iblk16n)

        pltpu.make_async_copy(b_h.at[pl.ds(0, L)], b_blk, bsem).wait()
        b16 = plsc.load_gather(b_blk, [iota, bcol16])
        o_v[pl.ds(g * L, L)] = 1.0 / (1.0 + jnp.exp(b16 - s16))

        @pl.when(not_last)
        def _():
            issue_b(jnp.right_shift(i16n, 7))

        return 0

    lax.fori_loop(0, GROUPS, group, 0)
    pltpu.sync_copy(o_v, out_h.at[pl.ds(base, BPW)])


@jax.jit
def _mirt_sc(user, item, theta_t, a_t, b128):
    mesh = plsc.VectorSubcoreMesh(
        core_axis_name="c", subcore_axis_name="s", num_cores=NC, num_subcores=NS
    )
    f = pl.kernel(
        _body,
        out_type=jax.ShapeDtypeStruct((B,), jnp.float32),
        mesh=mesh,
        scratch_types=[
            pltpu.VMEM((BPW,), jnp.int32),
            pltpu.VMEM((BPW,), jnp.int32),
            pltpu.VMEM((2, SG, D, 128), jnp.float32),
            pltpu.VMEM((2, SG, D, 128), jnp.float32),
            pltpu.VMEM((L, 128), jnp.float32),
            pltpu.VMEM((BPW,), jnp.float32),
            pltpu.SemaphoreType.DMA((2, SG)),
            pltpu.SemaphoreType.DMA((2, SG)),
            pltpu.SemaphoreType.DMA,
        ],
        compiler_params=pltpu.CompilerParams(needs_layout_passes=False),
    )
    return f(user, item, theta_t, a_t, b128)


def kernel(user, item, theta_w, a_w, b_w):
    b_flat = jnp.reshape(b_w, (-1,))
    npad = (-b_flat.shape[0]) % 128
    b128 = jnp.reshape(
        jnp.concatenate([b_flat, jnp.zeros((npad,), b_flat.dtype)]), (-1, 128)
    )
    return _mirt_sc(user, item, theta_w.T, a_w.T, b128)


# speedup vs baseline: 1.3585x; 1.0103x over previous
import jax
import jax.numpy as jnp
from jax import lax
from jax.experimental import pallas as pl
from jax.experimental.pallas import tpu as pltpu
from jax.experimental.pallas import tpu_sc as plsc

B = 16384
D = 32
NC = 2
NS = 16
NW = NC * NS
BPW = B // NW
L = 16
SG = 4
GROUPS = BPW // L

_LN2 = 0.6931471805599453
_C1 = _LN2 / (1 << 23)
_C2 = 126.94269504 * _LN2


def _softplus16(x):
    w = jnp.exp(-jnp.abs(x))
    z = 1.0 + w
    zb = plsc.bitcast(z, jnp.int32)
    y0 = zb.astype(jnp.float32) * _C1 - _C2
    t = z * jnp.exp(-y0) - 1.0
    corr = t * (1.0 + t * (-0.5 + t * (1.0 / 3.0 + t * -0.25)))
    return jnp.maximum(x, 0.0) + y0 + corr


def _body(user_h, item_h, theta_h, a_h, b_h, out_h,
          uidx_v, iidx_v, th_blk, a_blk, b_blk, o_v, sem, asem, bsem):
    wid = lax.axis_index("s") * NC + lax.axis_index("c")
    base = wid * BPW
    pltpu.sync_copy(user_h.at[pl.ds(base, BPW)], uidx_v)
    pltpu.sync_copy(item_h.at[pl.ds(base, BPW)], iidx_v)

    iota = lax.iota(jnp.int32, L)

    def issue(k, ublk16, iblk16):
        for j in range(SG):
            e = k * SG + j
            ub = ublk16[e]
            ib = iblk16[e]
            for r in range(4):
                pltpu.async_copy(
                    theta_h.at[pl.ds(8 * r, 8),
                               pl.ds(pl.multiple_of(ub * 128, 128), 128)],
                    th_blk.at[k % 2, j, pl.ds(8 * r, 8)], sem.at[k % 2, j])
                pltpu.async_copy(
                    a_h.at[pl.ds(8 * r, 8),
                           pl.ds(pl.multiple_of(ib * 128, 128), 128)],
                    a_blk.at[k % 2, j, pl.ds(8 * r, 8)], asem.at[k % 2, j])

    def issue_b(bq16):
        pltpu.async_copy(b_h.at[bq16], b_blk, bsem)

    def wait_sg(k):
        for j in range(SG):
            pltpu.make_async_copy(
                theta_h.at[:, pl.ds(0, 128)], th_blk.at[k % 2, j],
                sem.at[k % 2, j]).wait()
            pltpu.make_async_copy(
                a_h.at[:, pl.ds(0, 128)], a_blk.at[k % 2, j],
                asem.at[k % 2, j]).wait()

    def load_vecs(g):
        u16 = uidx_v[pl.ds(g * L, L)]
        i16 = iidx_v[pl.ds(g * L, L)]
        return u16, i16

    u16_0, i16_0 = load_vecs(0)
    issue(0, jnp.right_shift(u16_0, 7), jnp.right_shift(i16_0, 7))
    issue(1, jnp.right_shift(u16_0, 7), jnp.right_shift(i16_0, 7))
    issue_b(jnp.right_shift(i16_0, 7))

    def group(g, _):
        u16, i16 = load_vecs(g)
        ublk16 = jnp.right_shift(u16, 7)
        iblk16 = jnp.right_shift(i16, 7)
        ucol16 = jnp.bitwise_and(u16, 127)
        icol16 = jnp.bitwise_and(i16, 127)
        bcol16 = jnp.bitwise_and(i16, 127)
        gn = jnp.minimum(g + 1, GROUPS - 1)
        u16n, i16n = load_vecs(gn)
        ublk16n = jnp.right_shift(u16n, 7)
        iblk16n = jnp.right_shift(i16n, 7)
        not_last = g < GROUPS - 1

        def compute(k, acc):
            for j in range(SG):
                e = k * SG + j
                uc = jnp.full((L,), 1, jnp.int32) * ucol16[e]
                ic = jnp.full((L,), 1, jnp.int32) * icol16[e]
                th_lo = plsc.load_gather(th_blk.at[k % 2, j], [iota, uc])
                th_hi = plsc.load_gather(th_blk.at[k % 2, j], [iota + L, uc])
                a_lo = plsc.load_gather(a_blk.at[k % 2, j], [iota, ic])
                a_hi = plsc.load_gather(a_blk.at[k % 2, j], [iota + L, ic])
                val = _softplus16(a_lo) * th_lo + _softplus16(a_hi) * th_hi
                acc = jnp.where(iota == e, jnp.sum(val), acc)
            return acc

        s16 = jnp.zeros((L,), jnp.float32)
        wait_sg(0)
        s16 = compute(0, s16)
        issue(2, ublk16, iblk16)
        wait_sg(1)
        s16 = compute(1, s16)
        issue(3, ublk16, iblk16)
        wait_sg(2)
        s16 = compute(2, s16)

        @pl.when(not_last)
        def _():
            issue(0, ublk16n, iblk16n)

        wait_sg(3)
        s16 = compute(3, s16)

        @pl.when(not_last)
        def _():
            issue(1, ublk16n, iblk16n)

        pltpu.make_async_copy(b_h.at[pl.ds(0, L)], b_blk, bsem).wait()
        b16 = plsc.load_gather(b_blk, [iota, bcol16])
        o_v[pl.ds(g * L, L)] = 1.0 / (1.0 + jnp.exp(b16 - s16))

        @pl.when(not_last)
        def _():
            issue_b(jnp.right_shift(i16n, 7))

        return 0

    lax.fori_loop(0, GROUPS, group, 0)
    pltpu.sync_copy(o_v, out_h.at[pl.ds(base, BPW)])


@jax.jit
def _mirt_sc(user, item, theta_t, a_t, b128):
    mesh = plsc.VectorSubcoreMesh(
        core_axis_name="c", subcore_axis_name="s", num_cores=NC, num_subcores=NS
    )
    f = pl.kernel(
        _body,
        out_type=jax.ShapeDtypeStruct((B,), jnp.float32),
        mesh=mesh,
        scratch_types=[
            pltpu.VMEM((BPW,), jnp.int32),
            pltpu.VMEM((BPW,), jnp.int32),
            pltpu.VMEM((2, SG, D, 128), jnp.float32),
            pltpu.VMEM((2, SG, D, 128), jnp.float32),
            pltpu.VMEM((L, 128), jnp.float32),
            pltpu.VMEM((BPW,), jnp.float32),
            pltpu.SemaphoreType.DMA((2, SG)),
            pltpu.SemaphoreType.DMA((2, SG)),
            pltpu.SemaphoreType.DMA,
        ],
        compiler_params=pltpu.CompilerParams(needs_layout_passes=False),
    )
    return f(user, item, theta_t, a_t, b128)


def kernel(user, item, theta_w, a_w, b_w):
    b_flat = jnp.reshape(b_w, (-1,))
    npad = (-b_flat.shape[0]) % 128
    b128 = jnp.reshape(
        jnp.concatenate([b_flat, jnp.zeros((npad,), b_flat.dtype)]), (-1, 128)
    )
    return _mirt_sc(user, item, theta_w.T, a_w.T, b128)
